# Initial kernel scaffold; baseline (speedup 1.0000x reference)
import jax
import jax.numpy as jnp
from jax.experimental import pallas as pl


def kernel(x, edge_index, Wl1, bl1, Wr1, Wl2, bl2, Wr2, Wv, bv, Wt, bt):
    raise NotImplementedError("write your pallas kernel here")


# speedup vs baseline: 6.7063x; 6.7063x over previous
import jax
import jax.numpy as jnp
from jax import lax
from jax.experimental import pallas as pl
from jax.experimental.pallas import tpu as pltpu
from jax.experimental.pallas import tpu_sc as plsc

NC = 2
NS = 16
CH = 128


def _make_agg(n_nodes, dh, npad, m):
    rpt = npad // NS
    mesh = plsc.VectorSubcoreMesh(core_axis_name="c", subcore_axis_name="s")
    zchunks = [(k * CH, CH) for k in range(rpt // CH)]
    if rpt % CH:
        zchunks.append(((rpt // CH) * CH, rpt % CH))

    def body(hs_hbm, src_hbm, dst_hbm, zeros_hbm, out_hbm,
             src_v, dst_v, rows0, rows1, acc, semg0, semg1):
        c = lax.axis_index("c")
        s = lax.axis_index("s")
        hc_hbm = hs_hbm.at[c]

        pltpu.sync_copy(src_hbm.at[s], src_v)
        pltpu.sync_copy(dst_hbm.at[s], dst_v)
        for off, sz in zchunks:
            pltpu.sync_copy(zeros_hbm.at[pl.ds(0, sz)],
                            acc.at[pl.ds(s * rpt + off, sz)])
        plsc.subcore_barrier()

        def start_gather(j, buf, sem):
            pltpu.async_copy(hc_hbm.at[src_v.at[j]], buf, sem)

        start_gather(0, rows0, semg0)
        start_gather(1, rows1, semg1)

        def step(i, carry):
            a = 2 * i
            b = a + 1
            pltpu.make_async_copy(hc_hbm.at[src_v.at[a]], rows0, semg0).wait()
            pltpu.sync_copy(rows0, acc.at[dst_v.at[a]], add=True)

            @pl.when(a + 2 < m)
            def _():
                start_gather(a + 2, rows0, semg0)

            pltpu.make_async_copy(hc_hbm.at[src_v.at[b]], rows1, semg1).wait()
            pltpu.sync_copy(rows1, acc.at[dst_v.at[b]], add=True)

            @pl.when(b + 2 < m)
            def _():
                start_gather(b + 2, rows1, semg1)

            return carry

        lax.fori_loop(0, m // 2, step, 0)
        plsc.subcore_barrier()
        pltpu.sync_copy(acc.at[pl.ds(s * rpt, rpt)],
                        out_hbm.at[pl.ds(c * npad + s * rpt, rpt)])

    return pl.kernel(
        body,
        out_type=jax.ShapeDtypeStruct((NC * npad, dh), jnp.float32),
        compiler_params=pltpu.CompilerParams(use_tc_tiling_on_sc=False),
        mesh=mesh,
        scratch_types=[
            pltpu.VMEM((m, CH), jnp.int32),
            pltpu.VMEM((m, CH), jnp.int32),
            pltpu.VMEM((CH, dh), jnp.float32),
            pltpu.VMEM((CH, dh), jnp.float32),
            pltpu.VMEM_SHARED((npad, dh), jnp.float32),
            pltpu.SemaphoreType.DMA,
            pltpu.SemaphoreType.DMA,
        ])


def _make_deg(npad, m):
    rpt = npad // NS
    half = m // 2
    mesh = plsc.VectorSubcoreMesh(core_axis_name="c", subcore_axis_name="s")
    zchunks = [(k * CH, CH) for k in range(rpt // CH)]
    if rpt % CH:
        zchunks.append(((rpt // CH) * CH, rpt % CH))

    def body(dst_hbm, ones_hbm, zeros_hbm, out_hbm, dst_v, ones_v, dacc):
        c = lax.axis_index("c")
        s = lax.axis_index("s")
        pltpu.sync_copy(dst_hbm.at[s], dst_v)
        pltpu.sync_copy(ones_hbm, ones_v)
        for off, sz in zchunks:
            pltpu.sync_copy(zeros_hbm.at[pl.ds(0, sz)],
                            dacc.at[pl.ds(s * rpt + off, sz)])
        plsc.subcore_barrier()

        def step(i, carry):
            pltpu.sync_copy(ones_v, dacc.at[dst_v.at[c * half + i]], add=True)
            return carry

        lax.fori_loop(0, half, step, 0)
        plsc.subcore_barrier()
        pltpu.sync_copy(dacc.at[pl.ds(s * rpt, rpt)],
                        out_hbm.at[pl.ds(c * npad + s * rpt, rpt)])

    return pl.kernel(
        body,
        out_type=jax.ShapeDtypeStruct((NC * npad, 16), jnp.float32),
        compiler_params=pltpu.CompilerParams(use_tc_tiling_on_sc=False),
        mesh=mesh,
        scratch_types=[
            pltpu.VMEM((m, CH), jnp.int32),
            pltpu.VMEM((CH, 16), jnp.float32),
            pltpu.VMEM_SHARED((npad, 16), jnp.float32),
        ])


def _dgt(a, b):
    return lax.dot_general(a, b, (((1,), (1,)), ((), ())),
                           preferred_element_type=jnp.float32)


def _tc1_body(pa, pb, d0, d1, x, wl, bl, wr, o):
    deg = jnp.maximum(d0[:, :1] + d1[:, :1], 1.0)
    mean = jnp.concatenate([pa[...], pb[...]], axis=1) / deg
    o[...] = jnp.maximum(_dgt(mean, wl[...]) + bl[...] + _dgt(x[...], wr[...]),
                         0.0)


def _tc2_body(pa, pb, d0, d1, h1, wl, bl, wr, wv, bv, wt, bt, oh, ov, ot):
    deg = jnp.maximum(d0[:, :1] + d1[:, :1], 1.0)
    mean = jnp.concatenate([pa[...], pb[...]], axis=1) / deg
    h = _dgt(mean, wl[...]) + bl[...] + _dgt(h1[...], wr[...])
    oh[...] = h
    ov[...] = jnp.maximum(_dgt(h, wv[...]) + bv[...], 0.0)
    ot[...] = jnp.maximum(_dgt(h, wt[...]) + bt[...], 0.0)


def _row_spec(bn, w):
    return pl.BlockSpec((bn, w), lambda i: (i, 0))


def _full_spec():
    return pl.BlockSpec((128, 128), lambda i: (0, 0))


def _bias_spec():
    return pl.BlockSpec((1, 128), lambda i: (0, 0))


def kernel(x, edge_index, Wl1, bl1, Wr1, Wl2, bl2, Wr2, Wv, bv, Wt, bt):
    n, d = x.shape
    dh = d // 2
    e = edge_index.shape[1]
    m = -(-e // (NS * CH))
    m += m % 2
    ep = NS * m * CH
    npad = -(-(n + 1) // (NS * 8)) * (NS * 8)

    pad = ep - e
    srcp = jnp.concatenate(
        [edge_index[0], jnp.zeros((pad,), jnp.int32)]).reshape(NS, m, CH)
    dstp = jnp.concatenate(
        [edge_index[1], jnp.full((pad,), n, jnp.int32)]).reshape(NS, m, CH)
    zeros = jnp.zeros((CH, dh), jnp.float32)
    ones16 = jnp.ones((CH, 16), jnp.float32)
    zeros16 = jnp.zeros((CH, 16), jnp.float32)

    agg = _make_agg(n, dh, npad, m)
    deg = _make_deg(npad, m)

    degp = deg(dstp, ones16, zeros16)
    d0, d1 = degp[:n], degp[npad:npad + n]

    xs = jnp.stack([x[:, :dh], x[:, dh:]])
    agg1 = agg(xs, srcp, dstp, zeros)
    pa, pb = agg1[:n], agg1[npad:npad + n]

    bn = 1000
    grid = (n // bn,)
    h1 = pl.pallas_call(
        _tc1_body,
        grid=grid,
        in_specs=[_row_spec(bn, dh), _row_spec(bn, dh),
                  _row_spec(bn, 16), _row_spec(bn, 16),
                  _row_spec(bn, d), _full_spec(), _bias_spec(), _full_spec()],
        out_specs=_row_spec(bn, d),
        out_shape=jax.ShapeDtypeStruct((n, d), jnp.float32),
    )(pa, pb, d0, d1, x, Wl1, bl1.reshape(1, d), Wr1)

    h1s = jnp.stack([h1[:, :dh], h1[:, dh:]])
    agg2 = agg(h1s, srcp, dstp, zeros)
    qa, qb = agg2[:n], agg2[npad:npad + n]

    h, xv, xt = pl.pallas_call(
        _tc2_body,
        grid=grid,
        in_specs=[_row_spec(bn, dh), _row_spec(bn, dh),
                  _row_spec(bn, 16), _row_spec(bn, 16),
                  _row_spec(bn, d),
                  _full_spec(), _bias_spec(), _full_spec(),
                  _full_spec(), _bias_spec(),
                  _full_spec(), _bias_spec()],
        out_specs=[_row_spec(bn, d), _row_spec(bn, d), _row_spec(bn, d)],
        out_shape=[jax.ShapeDtypeStruct((n, d), jnp.float32),
                   jax.ShapeDtypeStruct((n, d), jnp.float32),
                   jax.ShapeDtypeStruct((n, d), jnp.float32)],
    )(qa, qb, d0, d1, h1, Wl2, bl2.reshape(1, d), Wr2,
      Wv, bv.reshape(1, d), Wt, bt.reshape(1, d))

    return (h, xv, xt)
